# baseline (speedup 1.0000x reference)
import functools
import math

import jax
import jax.numpy as jnp
from jax import lax
from jax.experimental import pallas as pl
from jax.experimental.pallas import tpu as pltpu
from jax.experimental.pallas import tpu_sc as plsc

D_MODEL = 64
SCALE = math.sqrt(D_MODEL)

NC = 2
NS = 16
NW = NC * NS
LANES = 16

N_TOK = 4096
N_POS = 200
RPW = N_TOK // NW
VOCAB_PAIRS = 500000

_mesh = plsc.VectorSubcoreMesh(
    core_axis_name="c", subcore_axis_name="s", num_cores=NC, num_subcores=NS
)


@functools.partial(
    pl.kernel,
    out_type=jax.ShapeDtypeStruct((N_POS, D_MODEL, N_TOK), jnp.float32),
    mesh=_mesh,
    scratch_types=[
        pltpu.VMEM((N_POS, RPW), jnp.int32),
        pltpu.VMEM((RPW,), jnp.int32),
        pltpu.VMEM((RPW, 2 * D_MODEL), jnp.float32),
        pltpu.VMEM((D_MODEL, RPW), jnp.float32),
        pltpu.SemaphoreType.DMA,
    ],
    compiler_params=pltpu.CompilerParams(needs_layout_passes=False),
)
def _emb_lookup(xt_hbm, tp_hbm, out_hbm, idx_v, ip_v, pairs_v, obuf_v, sem):
    wid = lax.axis_index("s") * NC + lax.axis_index("c")
    base = wid * RPW
    pltpu.sync_copy(xt_hbm.at[:, pl.ds(base, RPW)], idx_v)

    lane = lax.iota(jnp.int32, LANES)

    def chunk_body(p, carry):
        @plsc.parallel_loop(0, RPW // LANES, unroll=4)
        def _pairs(k):
            v = idx_v[p, pl.ds(k * LANES, LANES)]
            ip_v[pl.ds(k * LANES, LANES)] = lax.shift_right_logical(v, 1)

        pltpu.async_copy(tp_hbm.at[ip_v], pairs_v, sem).wait()

        for k in range(RPW // LANES):
            idx16 = idx_v[p, pl.ds(k * LANES, LANES)]
            col0 = (idx16 & 1) * D_MODEL
            row16 = lane + (k * LANES)

            @plsc.parallel_loop(0, D_MODEL, unroll=4)
            def _tr(f):
                vals = plsc.load_gather(pairs_v, [row16, col0 + f])
                obuf_v[f, pl.ds(k * LANES, LANES)] = vals * SCALE

        pltpu.sync_copy(obuf_v, out_hbm.at[p, :, pl.ds(base, RPW)])
        return carry

    lax.fori_loop(0, N_POS, chunk_body, 0)


def kernel(x, table):
    xt = jnp.transpose(x.astype(jnp.int32))
    tp = table.reshape(VOCAB_PAIRS, 2 * D_MODEL)
    out = _emb_lookup(xt, tp)
    return jnp.transpose(out, (2, 0, 1))


# speedup vs baseline: 1.2339x; 1.2339x over previous
import functools
import math

import jax
import jax.numpy as jnp
from jax import lax
from jax.experimental import pallas as pl
from jax.experimental.pallas import tpu as pltpu
from jax.experimental.pallas import tpu_sc as plsc

D_MODEL = 64
SCALE = math.sqrt(D_MODEL)

NC = 2
NS = 16
NW = NC * NS
LANES = 16

N_TOK = 4096
N_POS = 200
RPW = N_TOK // NW
NB = 3
G = 2
MB = 2

_mesh = plsc.VectorSubcoreMesh(
    core_axis_name="c", subcore_axis_name="s", num_cores=NC, num_subcores=NS
)


@functools.partial(
    pl.kernel,
    out_type=jax.ShapeDtypeStruct((N_POS, D_MODEL, N_TOK), jnp.float32),
    mesh=_mesh,
    scratch_types=[
        pltpu.VMEM((N_POS, RPW), jnp.int32),
        pltpu.VMEM((NB, RPW, 2 * D_MODEL), jnp.float32),
        pltpu.VMEM((MB, D_MODEL, RPW), jnp.float32),
        pltpu.SemaphoreType.DMA((NB,)),
        pltpu.SemaphoreType.DMA((MB,)),
    ],
    compiler_params=pltpu.CompilerParams(needs_layout_passes=False),
)
def _emb_lookup(xt_hbm, tw_hbm, out_hbm, idx_v, rows_v, obuf_v, gsem, ssem):
    wid = lax.axis_index("s") * NC + lax.axis_index("c")
    base = wid * RPW
    pltpu.sync_copy(xt_hbm.at[:, pl.ds(base, RPW)], idx_v)

    lane = lax.iota(jnp.int32, LANES)
    rows16 = [lane + (k * LANES) for k in range(RPW // LANES)]

    def gather(p, b):
        return pltpu.make_async_copy(
            tw_hbm.at[idx_v.at[p]], rows_v.at[b], gsem.at[b]
        )

    def store(p, m):
        return pltpu.make_async_copy(
            obuf_v.at[m], out_hbm.at[p, :, pl.ds(base, RPW)], ssem.at[m]
        )

    for p in range(G):
        gather(p, p % NB).start()

    def chunk_body(p, carry):
        b = p % NB
        m = p % MB

        @pl.when(p + G < N_POS)
        def _launch():
            gather(p + G, (p + G) % NB).start()

        gather(p, b).wait()

        @pl.when(p >= MB)
        def _drain():
            store(p - MB, m).wait()

        @plsc.parallel_loop(0, D_MODEL, unroll=4)
        def _tr(f):
            col16 = lax.broadcast_in_dim(f, (LANES,), ())
            for k in range(RPW // LANES):
                vals = plsc.load_gather(rows_v, [
                    lax.broadcast_in_dim(b, (LANES,), ()), rows16[k], col16])
                obuf_v[m, f, pl.ds(k * LANES, LANES)] = vals

        store(p, m).start()
        return carry

    lax.fori_loop(0, N_POS, chunk_body, 0)

    for p in range(N_POS - MB, N_POS):
        store(p, p % MB).wait()


def kernel(x, table):
    xt = jnp.transpose(x.astype(jnp.int32))
    tw = jnp.concatenate([table, table], axis=1) * SCALE
    out = _emb_lookup(xt, tw)
    return jnp.transpose(out, (2, 0, 1))
